# baseline (speedup 1.0000x reference)
import jax
import jax.numpy as jnp
from jax.experimental import pallas as pl
from jax.experimental.pallas import tpu as pltpu


def _splayer_dma_kernel(wav_hbm, len_hbm, wav_out, len_out, sem_wav, sem_len):
    wav_cp = pltpu.make_async_copy(wav_hbm, wav_out, sem_wav)
    len_cp = pltpu.make_async_copy(len_hbm, len_out, sem_len)
    wav_cp.start()
    len_cp.start()
    wav_cp.wait()
    len_cp.wait()


def kernel(wav_batch, lengths):
    lengths_2d = jnp.asarray(lengths).astype(jnp.int32).reshape(1, lengths.shape[0])
    wav_out, len_out = pl.pallas_call(
        _splayer_dma_kernel,
        in_specs=[
            pl.BlockSpec(memory_space=pl.ANY),
            pl.BlockSpec(memory_space=pl.ANY),
        ],
        out_specs=[
            pl.BlockSpec(memory_space=pl.ANY),
            pl.BlockSpec(memory_space=pl.ANY),
        ],
        out_shape=[
            jax.ShapeDtypeStruct(wav_batch.shape, wav_batch.dtype),
            jax.ShapeDtypeStruct(lengths_2d.shape, jnp.int32),
        ],
        scratch_shapes=[pltpu.SemaphoreType.DMA, pltpu.SemaphoreType.DMA],
    )(wav_batch, lengths_2d)
    return wav_out, len_out.reshape(lengths.shape)


# speedup vs baseline: 12.2848x; 12.2848x over previous
import jax
import jax.numpy as jnp
from jax.experimental import pallas as pl


_BB = 4


def _splayer_kernel(wav_ref, len_ref, wav_out_ref, len_out_ref):
    wav_out_ref[...] = wav_ref[...]
    len_out_ref[...] = len_ref[...]


def kernel(wav_batch, lengths):
    b, t, f = wav_batch.shape
    lengths_2d = jnp.asarray(lengths).astype(jnp.int32).reshape(1, lengths.shape[0])
    wav_out, len_out = pl.pallas_call(
        _splayer_kernel,
        grid=(b // _BB,),
        in_specs=[
            pl.BlockSpec((_BB, t, f), lambda i: (i, 0, 0)),
            pl.BlockSpec(lengths_2d.shape, lambda i: (0, 0)),
        ],
        out_specs=[
            pl.BlockSpec((_BB, t, f), lambda i: (i, 0, 0)),
            pl.BlockSpec(lengths_2d.shape, lambda i: (0, 0)),
        ],
        out_shape=[
            jax.ShapeDtypeStruct(wav_batch.shape, wav_batch.dtype),
            jax.ShapeDtypeStruct(lengths_2d.shape, jnp.int32),
        ],
    )(wav_batch, lengths_2d)
    return wav_out, len_out.reshape(lengths.shape)
